# Initial kernel scaffold; baseline (speedup 1.0000x reference)
import jax
import jax.numpy as jnp
from jax.experimental import pallas as pl


def kernel(cls_logits, box_pred, labels, boxes):
    raise NotImplementedError("write your pallas kernel here")


# speedup vs baseline: 2.5361x; 2.5361x over previous
import jax
import jax.numpy as jnp
from jax.experimental import pallas as pl
from jax.experimental.pallas import tpu as pltpu

_B, _C, _Hf, _Wf, _N = 16, 81, 32, 32, 32
_HW = _Hf * _Wf
_BG = 80
_CLS_WEIGHT = 1.0
_BOX_WEIGHT = 5.0


def _loss_kernel(cl_ref, bp_ref, bxn_ref, bxt_ref, lab_ref,
                 out_total, out_cls, out_box, acc):
    b = pl.program_id(0)

    @pl.when(b == 0)
    def _():
        acc[0] = 0.0
        acc[1] = 0.0
        acc[2] = 0.0
        acc[3] = 0.0

    cl = cl_ref[0]
    bp = bp_ref[0]
    bxn = bxn_ref[0]
    bxt = bxt_ref[0]
    lab = lab_ref[0]

    m = jnp.max(cl, axis=0, keepdims=True)
    ex = jnp.exp(cl - m)
    s = jnp.sum(ex, axis=0, keepdims=True)
    lse = m + jnp.log(s)
    sum_lse = jnp.sum(lse)
    bg_sum = jnp.sum(cl[_BG:_BG + 1, :])

    cx_s = (bxn[:, 0:1] + bxn[:, 2:3]) * (0.5 * _Wf)
    cy_s = (bxn[:, 1:2] + bxn[:, 3:4]) * (0.5 * _Hf)
    jj_s = jnp.floor(cx_s).astype(jnp.int32)
    ii_s = jnp.floor(cy_s).astype(jnp.int32)
    valid_s = (ii_s >= 0) & (ii_s < _Hf) & (jj_s >= 0) & (jj_s < _Wf)
    cell_s = ii_s * _Wf + jj_s

    cx_l = (bxt[0:1, :] + bxt[2:3, :]) * (0.5 * _Wf)
    cy_l = (bxt[1:2, :] + bxt[3:4, :]) * (0.5 * _Hf)
    jj_l = jnp.floor(cx_l).astype(jnp.int32)
    ii_l = jnp.floor(cy_l).astype(jnp.int32)
    valid_l = (ii_l >= 0) & (ii_l < _Hf) & (jj_l >= 0) & (jj_l < _Wf)
    cell_l = ii_l * _Wf + jj_l

    row = jax.lax.broadcasted_iota(jnp.int32, (_N, _N), 0)
    col = jax.lax.broadcasted_iota(jnp.int32, (_N, _N), 1)
    lose = (cell_s == cell_l) & (col > row) & valid_l
    n_later = jnp.sum(lose.astype(jnp.float32), axis=1, keepdims=True)
    winner = valid_s & (n_later == 0.0)
    wf_ = winner.astype(jnp.float32)
    n_obj = jnp.sum(wf_)

    kio = jax.lax.broadcasted_iota(jnp.int32, (_N, _HW), 1)
    hw1 = ((kio == cell_s) & winner).astype(jnp.float32)

    cio = jax.lax.broadcasted_iota(jnp.int32, (_N, _C), 1)
    mc = wf_ * ((cio == lab).astype(jnp.float32)
                - (cio == _BG).astype(jnp.float32))
    sel = jax.lax.dot_general(mc, hw1, (((0,), (0,)), ((), ())),
                              preferred_element_type=jnp.float32)
    corr = jnp.sum(sel * cl)

    box_num = 0.0
    for k in range(4):
        g = jnp.sum(hw1 * bp[k:k + 1, :], axis=1, keepdims=True)
        d = g - bxn[:, k:k + 1]
        ad = jnp.abs(d)
        sl1 = jnp.where(ad < 1.0, 0.5 * d * d, ad - 0.5)
        box_num = box_num + jnp.sum(wf_ * sl1)

    acc[0] += sum_lse
    acc[1] += bg_sum + corr
    acc[2] += box_num
    acc[3] += n_obj

    @pl.when(b == _B - 1)
    def _():
        loss_cls = (acc[0] - acc[1]) / (_B * _HW)
        nob = acc[3]
        denom = jnp.maximum(nob * 4.0, 1.0)
        loss_box = jnp.where(nob > 0.0, acc[2] / denom, 0.0)
        total = _CLS_WEIGHT * loss_cls + _BOX_WEIGHT * loss_box
        out_total[:, :] = jnp.full((1, 1), total, jnp.float32)
        out_cls[:, :] = jnp.full((1, 1), loss_cls, jnp.float32)
        out_box[:, :] = jnp.full((1, 1), loss_box, jnp.float32)


def kernel(cls_logits, box_pred, labels, boxes):
    cl3 = cls_logits.reshape(_B, _C, _HW)
    bp3 = box_pred.reshape(_B, 4, _HW)
    bxt = jnp.transpose(boxes, (0, 2, 1))
    lab3 = labels.reshape(_B, _N, 1)
    total, lcls, lbox = pl.pallas_call(
        _loss_kernel,
        grid=(_B,),
        in_specs=[
            pl.BlockSpec((1, _C, _HW), lambda b: (b, 0, 0)),
            pl.BlockSpec((1, 4, _HW), lambda b: (b, 0, 0)),
            pl.BlockSpec((1, _N, 4), lambda b: (b, 0, 0)),
            pl.BlockSpec((1, 4, _N), lambda b: (b, 0, 0)),
            pl.BlockSpec((1, _N, 1), lambda b: (b, 0, 0)),
        ],
        out_specs=[
            pl.BlockSpec((1, 1), lambda b: (0, 0)),
            pl.BlockSpec((1, 1), lambda b: (0, 0)),
            pl.BlockSpec((1, 1), lambda b: (0, 0)),
        ],
        out_shape=[
            jax.ShapeDtypeStruct((1, 1), jnp.float32),
            jax.ShapeDtypeStruct((1, 1), jnp.float32),
            jax.ShapeDtypeStruct((1, 1), jnp.float32),
        ],
        scratch_shapes=[pltpu.SMEM((4,), jnp.float32)],
    )(cl3, bp3, boxes, bxt, lab3)
    return (total[0, 0], lcls[0, 0], lbox[0, 0])


# speedup vs baseline: 3.6177x; 1.4265x over previous
import jax
import jax.numpy as jnp
from jax.experimental import pallas as pl
from jax.experimental.pallas import tpu as pltpu

_B, _C, _Hf, _Wf, _N = 16, 81, 32, 32, 32
_HW = _Hf * _Wf
_BG = 80
_CLS_WEIGHT = 1.0
_BOX_WEIGHT = 5.0


def _loss_kernel(cl_ref, bp_ref, bxn_ref, bxt_ref, lab_ref,
                 out_total, out_cls, out_box):
    cl = cl_ref[...]
    bp = bp_ref[...]
    bxn = bxn_ref[...]
    bxt = bxt_ref[...]
    lab = lab_ref[...]

    m = jnp.max(cl, axis=1, keepdims=True)
    s = jnp.sum(jnp.exp(cl - m), axis=1, keepdims=True)
    sum_lse = jnp.sum(m + jnp.log(s))
    bg_sum = jnp.sum(cl[:, _BG:_BG + 1, :])

    cx_s = (bxn[:, :, 0:1] + bxn[:, :, 2:3]) * (0.5 * _Wf)
    cy_s = (bxn[:, :, 1:2] + bxn[:, :, 3:4]) * (0.5 * _Hf)
    jj_s = jnp.floor(cx_s).astype(jnp.int32)
    ii_s = jnp.floor(cy_s).astype(jnp.int32)
    valid_s = (ii_s >= 0) & (ii_s < _Hf) & (jj_s >= 0) & (jj_s < _Wf)
    cell_s = ii_s * _Wf + jj_s

    cx_l = (bxt[:, 0:1, :] + bxt[:, 2:3, :]) * (0.5 * _Wf)
    cy_l = (bxt[:, 1:2, :] + bxt[:, 3:4, :]) * (0.5 * _Hf)
    jj_l = jnp.floor(cx_l).astype(jnp.int32)
    ii_l = jnp.floor(cy_l).astype(jnp.int32)
    valid_l = (ii_l >= 0) & (ii_l < _Hf) & (jj_l >= 0) & (jj_l < _Wf)
    cell_l = ii_l * _Wf + jj_l

    row = jax.lax.broadcasted_iota(jnp.int32, (_B, _N, _N), 1)
    col = jax.lax.broadcasted_iota(jnp.int32, (_B, _N, _N), 2)
    lose = (cell_s == cell_l) & (col > row) & valid_l
    n_later = jnp.sum(lose.astype(jnp.float32), axis=2, keepdims=True)
    winner = valid_s & (n_later == 0.0)
    wf_ = winner.astype(jnp.float32)
    n_obj = jnp.sum(wf_)

    kio = jax.lax.broadcasted_iota(jnp.int32, (_B, _N, _HW), 2)
    hw1 = ((kio == cell_s) & winner).astype(jnp.float32)

    gc = jax.lax.dot_general(hw1, cl, (((2,), (2,)), ((0,), (0,))),
                             preferred_element_type=jnp.float32)
    gb = jax.lax.dot_general(hw1, bp, (((2,), (2,)), ((0,), (0,))),
                             preferred_element_type=jnp.float32)

    cio = jax.lax.broadcasted_iota(jnp.int32, (_B, _N, _C), 2)
    pick = (cio == lab).astype(jnp.float32) - (cio == _BG).astype(jnp.float32)
    corr = jnp.sum(gc * pick)

    d = gb - bxn
    ad = jnp.abs(d)
    sl1 = jnp.where(ad < 1.0, 0.5 * d * d, ad - 0.5)
    box_num = jnp.sum(wf_ * sl1)

    loss_cls = (sum_lse - bg_sum - corr) / (_B * _HW)
    denom = jnp.maximum(n_obj * 4.0, 1.0)
    loss_box = jnp.where(n_obj > 0.0, box_num / denom, 0.0)
    total = _CLS_WEIGHT * loss_cls + _BOX_WEIGHT * loss_box
    out_total[:, :] = jnp.full((1, 1), total, jnp.float32)
    out_cls[:, :] = jnp.full((1, 1), loss_cls, jnp.float32)
    out_box[:, :] = jnp.full((1, 1), loss_box, jnp.float32)


def kernel(cls_logits, box_pred, labels, boxes):
    cl3 = cls_logits.reshape(_B, _C, _HW)
    bp3 = box_pred.reshape(_B, 4, _HW)
    bxt = jnp.transpose(boxes, (0, 2, 1))
    lab3 = labels.reshape(_B, _N, 1)
    total, lcls, lbox = pl.pallas_call(
        _loss_kernel,
        out_shape=[
            jax.ShapeDtypeStruct((1, 1), jnp.float32),
            jax.ShapeDtypeStruct((1, 1), jnp.float32),
            jax.ShapeDtypeStruct((1, 1), jnp.float32),
        ],
    )(cl3, bp3, boxes, bxt, lab3)
    return (total[0, 0], lcls[0, 0], lbox[0, 0])
